# baseline (speedup 1.0000x reference)
import functools

import jax
import jax.numpy as jnp
from jax import lax
from jax.experimental import pallas as pl
from jax.experimental.pallas import tpu as pltpu


def _round_up(x, m):
    return ((x + m - 1) // m) * m


def _lstm_kernel(xf_ref, xb_ref, wih_f_ref, whh_f_ref, b_f_ref,
                 wih_b_ref, whh_b_ref, b_b_ref, ef_ref, eb_ref,
                 xpf, xpb, hf_s, cf_s, hb_s, cb_s,
                 *, tc, batch, hidden):
    Bp, H = batch, hidden
    j = pl.program_id(0)

    wih_f = wih_f_ref[...].astype(jnp.bfloat16)
    wih_b = wih_b_ref[...].astype(jnp.bfloat16)
    whh_f = whh_f_ref[...].astype(jnp.bfloat16)
    whh_b = whh_b_ref[...].astype(jnp.bfloat16)

    @pl.when(j == 0)
    def _():
        z = jnp.zeros((Bp, H), jnp.float32)
        hf_s[...] = z
        cf_s[...] = z
        hb_s[...] = z
        cb_s[...] = z

    xpf[...] = (jnp.dot(xf_ref[...], wih_f,
                        preferred_element_type=jnp.float32) + b_f_ref[...])
    xpb[...] = (jnp.dot(xb_ref[...], wih_b,
                        preferred_element_type=jnp.float32) + b_b_ref[...])

    def cell(pre, c):
        i = jax.nn.sigmoid(pre[:, 0:H])
        f = jax.nn.sigmoid(pre[:, H:2 * H])
        g = jnp.tanh(pre[:, 2 * H:3 * H])
        o = jax.nn.sigmoid(pre[:, 3 * H:4 * H])
        c = f * c + i * g
        return o * jnp.tanh(c), c

    hf, cf, hb, cb = hf_s[...], cf_s[...], hb_s[...], cb_s[...]
    for tl in range(tc):
        rf = tl * Bp
        rb = (tc - 1 - tl) * Bp
        pre_f = xpf[pl.ds(rf, Bp), :] + jnp.dot(
            hf.astype(jnp.bfloat16), whh_f, preferred_element_type=jnp.float32)
        hf, cf = cell(pre_f, cf)
        ef_ref[pl.ds(rf, Bp), :] = hf.astype(jnp.bfloat16)
        pre_b = xpb[pl.ds(rb, Bp), :] + jnp.dot(
            hb.astype(jnp.bfloat16), whh_b, preferred_element_type=jnp.float32)
        hb, cb = cell(pre_b, cb)
        eb_ref[pl.ds(rb, Bp), :] = hb.astype(jnp.bfloat16)
    hf_s[...] = hf
    cf_s[...] = cf
    hb_s[...] = hb
    cb_s[...] = cb


def _head_kernel(ef_ref, eb_ref, gum_ref, wb_ref, bb_ref, wd_ref,
                 il_ref, lg_ref, *, inv_temp, n_gumbel, gp):
    T, BT, H = ef_ref.shape
    ef = jnp.transpose(ef_ref[...], (1, 0, 2)).reshape(BT * T, H)
    eb = jnp.transpose(eb_ref[...], (1, 0, 2)).reshape(BT * T, H)
    wb = wb_ref[...].astype(jnp.bfloat16)
    il = (jnp.dot(ef, wb[0:H], preferred_element_type=jnp.float32)
          + jnp.dot(eb, wb[H:2 * H], preferred_element_type=jnp.float32)
          + bb_ref[...])
    il_ref[...] = il
    y = (il + gum_ref[...]) * inv_temp
    if n_gumbel < gp:
        lane = lax.broadcasted_iota(jnp.int32, y.shape, 1)
        y = jnp.where(lane < n_gumbel, y, jnp.float32(-1e30))
    m = jnp.max(y, axis=-1, keepdims=True)
    e = jnp.exp(y - m)
    s = jnp.sum(e, axis=-1, keepdims=True)
    enc = e * pl.reciprocal(s, approx=True)
    lg_ref[...] = jnp.dot(enc.astype(jnp.bfloat16),
                          wd_ref[...].astype(jnp.bfloat16),
                          preferred_element_type=jnp.float32)


def kernel(x, wih_f, whh_f, b_f, wih_b, whh_b, b_b, wb, bias_b, wd,
           gumbel_noise):
    B, F, T = x.shape
    H = whh_f.shape[0]
    G = wb.shape[-1]
    C = wd.shape[-1]
    Bp = _round_up(max(B, 8), 8)
    Gp = _round_up(max(G, 128), 128)
    Cp = _round_up(max(C, 128), 128)
    TBp = T * Bp

    x_tbf = jnp.transpose(x.astype(jnp.bfloat16), (2, 0, 1))
    x_tbf = jnp.pad(x_tbf, ((0, 0), (0, Bp - B), (0, 0)))
    x_2d = x_tbf.reshape(TBp, F)

    TC = 8
    NCH = T // TC
    CR = TC * Bp

    lstm = functools.partial(_lstm_kernel, tc=TC, batch=Bp, hidden=H)
    wmap1 = lambda j: (0, 0)
    ef_tm, eb_tm = pl.pallas_call(
        lstm,
        grid=(NCH,),
        out_shape=(jax.ShapeDtypeStruct((TBp, H), jnp.bfloat16),
                   jax.ShapeDtypeStruct((TBp, H), jnp.bfloat16)),
        in_specs=[
            pl.BlockSpec((CR, F), lambda j: (j, 0)),
            pl.BlockSpec((CR, F), lambda j: (NCH - 1 - j, 0)),
            pl.BlockSpec((F, 4 * H), wmap1),
            pl.BlockSpec((H, 4 * H), wmap1),
            pl.BlockSpec((1, 4 * H), wmap1),
            pl.BlockSpec((F, 4 * H), wmap1),
            pl.BlockSpec((H, 4 * H), wmap1),
            pl.BlockSpec((1, 4 * H), wmap1),
        ],
        out_specs=(pl.BlockSpec((CR, H), lambda j: (j, 0)),
                   pl.BlockSpec((CR, H), lambda j: (NCH - 1 - j, 0))),
        scratch_shapes=[pltpu.VMEM((CR, 4 * H), jnp.float32),
                        pltpu.VMEM((CR, 4 * H), jnp.float32),
                        pltpu.VMEM((Bp, H), jnp.float32),
                        pltpu.VMEM((Bp, H), jnp.float32),
                        pltpu.VMEM((Bp, H), jnp.float32),
                        pltpu.VMEM((Bp, H), jnp.float32)],
        compiler_params=pltpu.CompilerParams(
            dimension_semantics=("arbitrary",)),
    )(x_2d, x_2d, wih_f, whh_f, b_f, wih_b, whh_b, b_b)

    ef_3d = ef_tm.reshape(T, Bp, H)
    eb_3d = eb_tm.reshape(T, Bp, H)

    gum_2d = gumbel_noise.reshape(B * T, G)
    gum_2d = jnp.pad(gum_2d, ((0, (Bp - B) * T), (0, Gp - G)))

    wb_p = jnp.pad(wb, ((0, 0), (0, Gp - G)))
    bb_p = jnp.pad(bias_b, ((0, 0), (0, Gp - G)))
    wd_p = jnp.pad(wd, ((0, Gp - G), (0, Cp - C)))

    BT = 8
    NBLK = Bp // BT
    R = BT * T
    row_map = lambda j: (j, 0)
    wmap = lambda j: (0, 0)
    emap = lambda j: (0, j, 0)
    head = functools.partial(_head_kernel, inv_temp=1.0, n_gumbel=G, gp=Gp)
    il2, lg2 = pl.pallas_call(
        head,
        grid=(NBLK,),
        out_shape=(jax.ShapeDtypeStruct((TBp, Gp), jnp.float32),
                   jax.ShapeDtypeStruct((TBp, Cp), jnp.float32)),
        in_specs=[
            pl.BlockSpec((T, BT, H), emap),
            pl.BlockSpec((T, BT, H), emap),
            pl.BlockSpec((R, Gp), row_map),
            pl.BlockSpec((2 * H, Gp), wmap),
            pl.BlockSpec((1, Gp), wmap),
            pl.BlockSpec((Gp, Cp), wmap),
        ],
        out_specs=(pl.BlockSpec((R, Gp), row_map),
                   pl.BlockSpec((R, Cp), row_map)),
        compiler_params=pltpu.CompilerParams(
            dimension_semantics=("arbitrary",)),
    )(ef_3d, eb_3d, gum_2d, wb_p, bb_p, wd_p)

    in_logit = il2.reshape(Bp, T, Gp)[:B, :, :G]
    logit = lg2.reshape(Bp, T, Cp)[:B, :, :C]
    return in_logit, logit


# speedup vs baseline: 1.0384x; 1.0384x over previous
import functools

import jax
import jax.numpy as jnp
from jax import lax
from jax.experimental import pallas as pl
from jax.experimental.pallas import tpu as pltpu


def _round_up(x, m):
    return ((x + m - 1) // m) * m


def _lstm_kernel(xf_ref, xb_ref, wih_f_ref, whh_f_ref, b_f_ref,
                 wih_b_ref, whh_b_ref, b_b_ref, ef_ref, eb_ref,
                 xpf, xpb, hf_s, cf_s, hb_s, cb_s,
                 *, tc, batch, hidden):
    Bp, H = batch, hidden
    j = pl.program_id(0)

    wih_f = wih_f_ref[...].astype(jnp.bfloat16)
    wih_b = wih_b_ref[...].astype(jnp.bfloat16)
    whh_f = whh_f_ref[...].astype(jnp.bfloat16)
    whh_b = whh_b_ref[...].astype(jnp.bfloat16)

    @pl.when(j == 0)
    def _():
        z = jnp.zeros((Bp, H), jnp.float32)
        hf_s[...] = z
        cf_s[...] = z
        hb_s[...] = z
        cb_s[...] = z

    xpf[...] = (jnp.dot(xf_ref[...], wih_f,
                        preferred_element_type=jnp.float32) + b_f_ref[...])
    xpb[...] = (jnp.dot(xb_ref[...], wih_b,
                        preferred_element_type=jnp.float32) + b_b_ref[...])

    def cell(pre, c):
        i = jax.nn.sigmoid(pre[:, 0:H])
        f = jax.nn.sigmoid(pre[:, H:2 * H])
        g = jnp.tanh(pre[:, 2 * H:3 * H])
        o = jax.nn.sigmoid(pre[:, 3 * H:4 * H])
        c = f * c + i * g
        return o * jnp.tanh(c), c

    hf, cf, hb, cb = hf_s[...], cf_s[...], hb_s[...], cb_s[...]
    for tl in range(tc):
        rf = tl * Bp
        rb = (tc - 1 - tl) * Bp
        pre_f = xpf[pl.ds(rf, Bp), :] + jnp.dot(
            hf.astype(jnp.bfloat16), whh_f, preferred_element_type=jnp.float32)
        hf, cf = cell(pre_f, cf)
        ef_ref[pl.ds(rf, Bp), :] = hf.astype(jnp.bfloat16)
        pre_b = xpb[pl.ds(rb, Bp), :] + jnp.dot(
            hb.astype(jnp.bfloat16), whh_b, preferred_element_type=jnp.float32)
        hb, cb = cell(pre_b, cb)
        eb_ref[pl.ds(rb, Bp), :] = hb.astype(jnp.bfloat16)
    hf_s[...] = hf
    cf_s[...] = cf
    hb_s[...] = hb
    cb_s[...] = cb


def _head_kernel(ef_ref, eb_ref, gum_ref, wb_ref, bb_ref, wd_ref,
                 il_ref, lg_ref, *, inv_temp, n_gumbel, gp):
    T, BT, H = ef_ref.shape
    ef = jnp.transpose(ef_ref[...], (1, 0, 2)).reshape(BT * T, H)
    eb = jnp.transpose(eb_ref[...], (1, 0, 2)).reshape(BT * T, H)
    wb = wb_ref[...].astype(jnp.bfloat16)
    il = (jnp.dot(ef, wb[0:H], preferred_element_type=jnp.float32)
          + jnp.dot(eb, wb[H:2 * H], preferred_element_type=jnp.float32)
          + bb_ref[...])
    il_ref[...] = il
    y = (il + gum_ref[...]) * inv_temp
    if n_gumbel < gp:
        lane = lax.broadcasted_iota(jnp.int32, y.shape, 1)
        y = jnp.where(lane < n_gumbel, y, jnp.float32(-1e30))
    m = jnp.max(y, axis=-1, keepdims=True)
    e = jnp.exp(y - m)
    s = jnp.sum(e, axis=-1, keepdims=True)
    enc = e * pl.reciprocal(s, approx=True)
    lg_ref[...] = jnp.dot(enc.astype(jnp.bfloat16),
                          wd_ref[...].astype(jnp.bfloat16),
                          preferred_element_type=jnp.float32)


def kernel(x, wih_f, whh_f, b_f, wih_b, whh_b, b_b, wb, bias_b, wd,
           gumbel_noise):
    B, F, T = x.shape
    H = whh_f.shape[0]
    G = wb.shape[-1]
    C = wd.shape[-1]
    Bp = _round_up(max(B, 8), 8)
    Gp = _round_up(max(G, 128), 128)
    Cp = _round_up(max(C, 128), 128)
    TBp = T * Bp

    x_tbf = jnp.transpose(x.astype(jnp.bfloat16), (2, 0, 1))
    x_tbf = jnp.pad(x_tbf, ((0, 0), (0, Bp - B), (0, 0)))
    x_2d = x_tbf.reshape(TBp, F)

    TC = 16
    NCH = T // TC
    CR = TC * Bp

    lstm = functools.partial(_lstm_kernel, tc=TC, batch=Bp, hidden=H)
    wmap1 = lambda j: (0, 0)
    ef_tm, eb_tm = pl.pallas_call(
        lstm,
        grid=(NCH,),
        out_shape=(jax.ShapeDtypeStruct((TBp, H), jnp.bfloat16),
                   jax.ShapeDtypeStruct((TBp, H), jnp.bfloat16)),
        in_specs=[
            pl.BlockSpec((CR, F), lambda j: (j, 0)),
            pl.BlockSpec((CR, F), lambda j: (NCH - 1 - j, 0)),
            pl.BlockSpec((F, 4 * H), wmap1),
            pl.BlockSpec((H, 4 * H), wmap1),
            pl.BlockSpec((1, 4 * H), wmap1),
            pl.BlockSpec((F, 4 * H), wmap1),
            pl.BlockSpec((H, 4 * H), wmap1),
            pl.BlockSpec((1, 4 * H), wmap1),
        ],
        out_specs=(pl.BlockSpec((CR, H), lambda j: (j, 0)),
                   pl.BlockSpec((CR, H), lambda j: (NCH - 1 - j, 0))),
        scratch_shapes=[pltpu.VMEM((CR, 4 * H), jnp.float32),
                        pltpu.VMEM((CR, 4 * H), jnp.float32),
                        pltpu.VMEM((Bp, H), jnp.float32),
                        pltpu.VMEM((Bp, H), jnp.float32),
                        pltpu.VMEM((Bp, H), jnp.float32),
                        pltpu.VMEM((Bp, H), jnp.float32)],
        compiler_params=pltpu.CompilerParams(
            dimension_semantics=("arbitrary",)),
    )(x_2d, x_2d, wih_f, whh_f, b_f, wih_b, whh_b, b_b)

    ef_3d = ef_tm.reshape(T, Bp, H)
    eb_3d = eb_tm.reshape(T, Bp, H)

    gum_2d = gumbel_noise.reshape(B * T, G)
    gum_2d = jnp.pad(gum_2d, ((0, (Bp - B) * T), (0, Gp - G)))

    wb_p = jnp.pad(wb, ((0, 0), (0, Gp - G)))
    bb_p = jnp.pad(bias_b, ((0, 0), (0, Gp - G)))
    wd_p = jnp.pad(wd, ((0, Gp - G), (0, Cp - C)))

    BT = 8
    NBLK = Bp // BT
    R = BT * T
    row_map = lambda j: (j, 0)
    wmap = lambda j: (0, 0)
    emap = lambda j: (0, j, 0)
    head = functools.partial(_head_kernel, inv_temp=1.0, n_gumbel=G, gp=Gp)
    il2, lg2 = pl.pallas_call(
        head,
        grid=(NBLK,),
        out_shape=(jax.ShapeDtypeStruct((TBp, Gp), jnp.float32),
                   jax.ShapeDtypeStruct((TBp, Cp), jnp.float32)),
        in_specs=[
            pl.BlockSpec((T, BT, H), emap),
            pl.BlockSpec((T, BT, H), emap),
            pl.BlockSpec((R, Gp), row_map),
            pl.BlockSpec((2 * H, Gp), wmap),
            pl.BlockSpec((1, Gp), wmap),
            pl.BlockSpec((Gp, Cp), wmap),
        ],
        out_specs=(pl.BlockSpec((R, Gp), row_map),
                   pl.BlockSpec((R, Cp), row_map)),
        compiler_params=pltpu.CompilerParams(
            dimension_semantics=("arbitrary",)),
    )(ef_3d, eb_3d, gum_2d, wb_p, bb_p, wd_p)

    in_logit = il2.reshape(Bp, T, Gp)[:B, :, :G]
    logit = lg2.reshape(Bp, T, Cp)[:B, :, :C]
    return in_logit, logit


# speedup vs baseline: 1.0644x; 1.0251x over previous
import functools

import jax
import jax.numpy as jnp
from jax import lax
from jax.experimental import pallas as pl
from jax.experimental.pallas import tpu as pltpu


def _round_up(x, m):
    return ((x + m - 1) // m) * m


def _lstm_kernel(x_ref, wih_f_ref, whh_f_ref, b_f_ref,
                 wih_b_ref, whh_b_ref, b_b_ref, ef_ref, eb_ref,
                 xpf, xpb, *, seq_len, batch, hidden):
    T, Bp, H = seq_len, batch, hidden

    wih_f = wih_f_ref[...].astype(jnp.bfloat16)
    wih_b = wih_b_ref[...].astype(jnp.bfloat16)
    whh_f = whh_f_ref[...].astype(jnp.bfloat16)
    whh_b = whh_b_ref[...].astype(jnp.bfloat16)

    x = x_ref[...]
    xpf[...] = (jnp.dot(x, wih_f,
                        preferred_element_type=jnp.float32) + b_f_ref[...])
    xpb[...] = (jnp.dot(x, wih_b,
                        preferred_element_type=jnp.float32) + b_b_ref[...])

    def cell(pre, c):
        i = jax.nn.sigmoid(pre[:, 0:H])
        f = jax.nn.sigmoid(pre[:, H:2 * H])
        g = jnp.tanh(pre[:, 2 * H:3 * H])
        o = jax.nn.sigmoid(pre[:, 3 * H:4 * H])
        c = f * c + i * g
        return o * jnp.tanh(c), c

    z = jnp.zeros((Bp, H), jnp.float32)
    hf, cf, hb, cb = z, z, z, z
    for t in range(T):
        rf = t * Bp
        rb = (T - 1 - t) * Bp
        pre_f = xpf[pl.ds(rf, Bp), :] + jnp.dot(
            hf.astype(jnp.bfloat16), whh_f, preferred_element_type=jnp.float32)
        hf, cf = cell(pre_f, cf)
        ef_ref[pl.ds(rf, Bp), :] = hf.astype(jnp.bfloat16)
        pre_b = xpb[pl.ds(rb, Bp), :] + jnp.dot(
            hb.astype(jnp.bfloat16), whh_b, preferred_element_type=jnp.float32)
        hb, cb = cell(pre_b, cb)
        eb_ref[pl.ds(rb, Bp), :] = hb.astype(jnp.bfloat16)


def _head_kernel(ef_ref, eb_ref, gum_ref, wb_ref, bb_ref, wd_ref,
                 il_ref, lg_ref, *, inv_temp, n_gumbel, gp):
    T, BT, H = ef_ref.shape
    ef = jnp.transpose(ef_ref[...], (1, 0, 2)).reshape(BT * T, H)
    eb = jnp.transpose(eb_ref[...], (1, 0, 2)).reshape(BT * T, H)
    wb = wb_ref[...].astype(jnp.bfloat16)
    il = (jnp.dot(ef, wb[0:H], preferred_element_type=jnp.float32)
          + jnp.dot(eb, wb[H:2 * H], preferred_element_type=jnp.float32)
          + bb_ref[...])
    il_ref[...] = il
    y = (il + gum_ref[...]) * inv_temp
    if n_gumbel < gp:
        lane = lax.broadcasted_iota(jnp.int32, y.shape, 1)
        y = jnp.where(lane < n_gumbel, y, jnp.float32(-1e30))
    m = jnp.max(y, axis=-1, keepdims=True)
    e = jnp.exp(y - m)
    s = jnp.sum(e, axis=-1, keepdims=True)
    enc = e * pl.reciprocal(s, approx=True)
    lg_ref[...] = jnp.dot(enc.astype(jnp.bfloat16),
                          wd_ref[...].astype(jnp.bfloat16),
                          preferred_element_type=jnp.float32)


def kernel(x, wih_f, whh_f, b_f, wih_b, whh_b, b_b, wb, bias_b, wd,
           gumbel_noise):
    B, F, T = x.shape
    H = whh_f.shape[0]
    G = wb.shape[-1]
    C = wd.shape[-1]
    Bp = _round_up(max(B, 8), 8)
    Gp = _round_up(max(G, 128), 128)
    Cp = _round_up(max(C, 128), 128)
    TBp = T * Bp

    x_tbf = jnp.transpose(x.astype(jnp.bfloat16), (2, 0, 1))
    x_tbf = jnp.pad(x_tbf, ((0, 0), (0, Bp - B), (0, 0)))
    x_2d = x_tbf.reshape(TBp, F)

    lstm = functools.partial(_lstm_kernel, seq_len=T, batch=Bp, hidden=H)
    wmap1 = lambda j: (0, 0)
    ef_tm, eb_tm = pl.pallas_call(
        lstm,
        grid=(1,),
        out_shape=(jax.ShapeDtypeStruct((TBp, H), jnp.bfloat16),
                   jax.ShapeDtypeStruct((TBp, H), jnp.bfloat16)),
        in_specs=[
            pl.BlockSpec((TBp, F), wmap1),
            pl.BlockSpec((F, 4 * H), wmap1),
            pl.BlockSpec((H, 4 * H), wmap1),
            pl.BlockSpec((1, 4 * H), wmap1),
            pl.BlockSpec((F, 4 * H), wmap1),
            pl.BlockSpec((H, 4 * H), wmap1),
            pl.BlockSpec((1, 4 * H), wmap1),
        ],
        out_specs=(pl.BlockSpec((TBp, H), wmap1),
                   pl.BlockSpec((TBp, H), wmap1)),
        scratch_shapes=[pltpu.VMEM((TBp, 4 * H), jnp.float32),
                        pltpu.VMEM((TBp, 4 * H), jnp.float32)],
        compiler_params=pltpu.CompilerParams(
            dimension_semantics=("arbitrary",)),
    )(x_2d, wih_f, whh_f, b_f, wih_b, whh_b, b_b)

    ef_3d = ef_tm.reshape(T, Bp, H)
    eb_3d = eb_tm.reshape(T, Bp, H)

    gum_2d = gumbel_noise.reshape(B * T, G)
    gum_2d = jnp.pad(gum_2d, ((0, (Bp - B) * T), (0, Gp - G)))

    wb_p = jnp.pad(wb, ((0, 0), (0, Gp - G)))
    bb_p = jnp.pad(bias_b, ((0, 0), (0, Gp - G)))
    wd_p = jnp.pad(wd, ((0, Gp - G), (0, Cp - C)))

    BT = 8
    NBLK = Bp // BT
    R = BT * T
    row_map = lambda j: (j, 0)
    wmap = lambda j: (0, 0)
    emap = lambda j: (0, j, 0)
    head = functools.partial(_head_kernel, inv_temp=1.0, n_gumbel=G, gp=Gp)
    il2, lg2 = pl.pallas_call(
        head,
        grid=(NBLK,),
        out_shape=(jax.ShapeDtypeStruct((TBp, Gp), jnp.float32),
                   jax.ShapeDtypeStruct((TBp, Cp), jnp.float32)),
        in_specs=[
            pl.BlockSpec((T, BT, H), emap),
            pl.BlockSpec((T, BT, H), emap),
            pl.BlockSpec((R, Gp), row_map),
            pl.BlockSpec((2 * H, Gp), wmap),
            pl.BlockSpec((1, Gp), wmap),
            pl.BlockSpec((Gp, Cp), wmap),
        ],
        out_specs=(pl.BlockSpec((R, Gp), row_map),
                   pl.BlockSpec((R, Cp), row_map)),
        compiler_params=pltpu.CompilerParams(
            dimension_semantics=("arbitrary",)),
    )(ef_3d, eb_3d, gum_2d, wb_p, bb_p, wd_p)

    in_logit = il2.reshape(Bp, T, Gp)[:B, :, :G]
    logit = lg2.reshape(Bp, T, Cp)[:B, :, :C]
    return in_logit, logit
